# Initial kernel scaffold; baseline (speedup 1.0000x reference)
import jax
import jax.numpy as jnp
from jax.experimental import pallas as pl


def kernel(x, params):
    raise NotImplementedError("write your pallas kernel here")


# speedup vs baseline: 2.8885x; 2.8885x over previous
import jax
import jax.numpy as jnp
from jax import lax
from jax.experimental import pallas as pl

B = 1024
LATENT = 256
MAX_NODES = 64
MAX_EDGES = 128
NEF = 8
NNF = 16
NUM_NODES = B * MAX_NODES
G = 8
VN = 72

_f32 = jnp.float32
_HI = lax.Precision.HIGHEST


def _flatten_params(params):
    ws = []
    for name in ('edge_mlp', 'feature_mlp', 'node_mlp'):
        for W, b in params[name]:
            ws.append(W)
            ws.append(b.reshape(1, -1))
    gs = []
    for name in ('prob_gnn', 'final_gnn'):
        p = params[name]
        for ep, npar in zip(p['edge'], p['node']):
            We, be = ep[0]
            Wn, bn = npar[0]
            gs.append((We, be.reshape(1, -1), Wn, bn.reshape(1, -1)))
    return ws, gs


def _bmm_hi(a, b):
    return lax.dot_general(a, b, (((2,), (1,)), ((0,), (0,))),
                           precision=_HI, preferred_element_type=_f32)


def _bmm_t_hi(a, b):
    return lax.dot_general(a, b, (((1,), (1,)), ((0,), (0,))),
                           precision=_HI, preferred_element_type=_f32)


def _dense(v, w, bias):
    g, m, k = v.shape
    r = jnp.dot(v.reshape(g * m, k), w[...],
                preferred_element_type=_f32) + bias[...]
    return r.reshape(g, m, r.shape[-1])


def _call(body, n_in_blocked, blocked_specs, weights, out_specs, out_shape):
    def wrap(*args):
        in_specs = list(blocked_specs)
        for w in args[n_in_blocked:]:
            in_specs.append(
                pl.BlockSpec(w.shape, lambda i, nd=w.ndim: (0,) * nd))
        return pl.pallas_call(
            body, grid=(B // G,), in_specs=in_specs,
            out_specs=out_specs, out_shape=out_shape)(*args)
    return wrap


def _mlp3_body(x_ref, w1e, b1e, w2e, b2e, w1f, b1f, w2f, b2f,
               w1n, b1n, w2n, b2n, lo, fo, no):
    x = x_ref[...]
    h = jnp.maximum(jnp.dot(x, w1e[...], preferred_element_type=_f32)
                    + b1e[...], 0.0)
    lo[...] = jnp.dot(h, w2e[...], preferred_element_type=_f32) + b2e[...]
    h = jnp.maximum(jnp.dot(x, w1f[...], preferred_element_type=_f32)
                    + b1f[...], 0.0)
    fo[...] = jnp.dot(h, w2f[...], preferred_element_type=_f32) + b2f[...]
    h = jnp.maximum(jnp.dot(x, w1n[...], preferred_element_type=_f32)
                    + b1n[...], 0.0)
    n = jnp.dot(h, w2n[...], preferred_element_type=_f32) + b2n[...]
    no[...] = n.reshape(G, MAX_NODES, NNF)


def _onehots(i1, i2):
    iota64 = lax.broadcasted_iota(jnp.int32, (G, MAX_EDGES, MAX_NODES), 2)
    oh1 = (iota64 == i1[:, :, None]).astype(_f32)
    oh2 = (iota64 == i2[:, :, None]).astype(_f32)
    return oh1, oh2


def _top2_body(soft_ref, feats_ref, nodes_ref, we, be,
               i1o, i2o, ieo, e1o):
    soft = soft_ref[...]
    iota64 = lax.broadcasted_iota(jnp.int32, (G, MAX_EDGES, MAX_NODES), 2)
    i1 = jnp.argmax(soft, axis=2)
    oh1 = (iota64 == i1[:, :, None]).astype(_f32)
    v13 = jnp.max(soft, axis=2, keepdims=True)
    soft2 = jnp.where(oh1 > 0, -1e30, soft)
    i2 = jnp.argmax(soft2, axis=2)
    oh2 = (iota64 == i2[:, :, None]).astype(_f32)
    v23 = jnp.max(soft2, axis=2, keepdims=True)
    i1o[...] = i1
    i2o[...] = i2
    feats = feats_ref[...].reshape(G, MAX_EDGES, NEF - 2)
    init_edges = jnp.concatenate([v13, v23, feats], axis=2)
    ieo[...] = init_edges.reshape(G * MAX_EDGES, NEF)
    nodes = nodes_ref[...]
    e_in = jnp.concatenate(
        [init_edges, _bmm_hi(oh1, nodes), _bmm_hi(oh2, nodes)], axis=2)
    e1o[...] = _dense(e_in, we, be).reshape(G * MAX_EDGES, -1)


def _round_body(nodes_ref, agg_ref, edges_ref, i1_ref, i2_ref,
                wn, bn, we, be, no, eo):
    n3 = _dense(jnp.concatenate([nodes_ref[...], agg_ref[...]], axis=2),
                wn, bn)
    no[...] = n3
    oh1, oh2 = _onehots(i1_ref[...], i2_ref[...])
    e = edges_ref[...].reshape(G, MAX_EDGES, -1)
    e_in = jnp.concatenate([e, _bmm_hi(oh1, n3), _bmm_hi(oh2, n3)], axis=2)
    eo[...] = _dense(e_in, we, be).reshape(G * MAX_EDGES, -1)


def _nodelast_body(nodes_ref, agg_ref, wn, bn, o):
    o[...] = _dense(jnp.concatenate([nodes_ref[...], agg_ref[...]], axis=2),
                    wn, bn)


def _final_body(s_ref, i1_ref, i2_ref, nodes_ref, edges_ref, nn_ref, ne_ref,
                fwe0, fbe0, fwn0, fbn0, fwe1, fbe1, fwn1, fbn1,
                nodes_out, edges_out, snd_out, rcv_out):
    s = s_ref[...]
    gt = s[:, None, :] > s[:, :, None]
    ii = lax.broadcasted_iota(jnp.int32, (G, 64, 64), 1)
    jj = lax.broadcasted_iota(jnp.int32, (G, 64, 64), 2)
    tie = (s[:, None, :] == s[:, :, None]) & (jj < ii)
    rank = jnp.sum((gt | tie).astype(_f32), axis=2)
    n_node = nn_ref[...][:, 0, 0]
    n_edge = ne_ref[...][:, 0, 0]
    init_nodes = nodes_ref[...]
    init_edges = edges_ref[...].reshape(G, MAX_EDGES, NEF)
    oh1, oh2 = _onehots(i1_ref[...], i2_ref[...])

    iota_r = lax.broadcasted_iota(jnp.int32, (G, 64, 64), 2).astype(_f32)
    perm = (iota_r == rank[:, :, None]).astype(_f32)
    sorted_nodes = _bmm_t_hi(perm, init_nodes)
    iota_n = lax.broadcasted_iota(jnp.int32, (G, 64, 1), 1).astype(_f32)
    sorted_nodes = sorted_nodes * (iota_n < n_node[:, None, None]).astype(_f32)

    ns = _bmm_hi(oh1, rank[:, :, None])[:, :, 0]
    nr = _bmm_hi(oh2, rank[:, :, None])[:, :, 0]
    logic = ((ns < n_node[:, None]) & (nr < n_node[:, None])).astype(_f32)
    n_inc = jnp.sum(logic, axis=1)

    ja = lax.broadcasted_iota(jnp.int32, (MAX_EDGES, MAX_EDGES), 0).astype(_f32)
    jb = lax.broadcasted_iota(jnp.int32, (MAX_EDGES, MAX_EDGES), 1).astype(_f32)
    c_inc_gt = jnp.dot(logic, (ja > jb).astype(_f32), precision=_HI,
                       preferred_element_type=_f32)
    c_exc_lt = jnp.dot(1.0 - logic, (ja < jb).astype(_f32), precision=_HI,
                       preferred_element_type=_f32)
    pos = jnp.where(logic > 0, c_inc_gt, n_inc[:, None] + c_exc_lt)
    iota_p = lax.broadcasted_iota(jnp.int32, (G, 128, 128), 2).astype(_f32)
    q = (iota_p == pos[:, :, None]).astype(_f32)
    sorted_edges = _bmm_t_hi(q, init_edges)
    ns_s = _bmm_t_hi(q, ns[:, :, None])[:, :, 0]
    nr_s = _bmm_t_hi(q, nr[:, :, None])[:, :, 0]

    n_edge_f = jnp.minimum(n_edge, n_inc)
    iota_e = lax.broadcasted_iota(jnp.int32, (G, 128), 1).astype(_f32)
    excl = iota_e >= n_edge_f[:, None]
    keep = (~excl).astype(_f32)
    sorted_edges = sorted_edges * keep[:, :, None]

    iota_v = lax.broadcasted_iota(jnp.int32, (G, 128, VN), 2).astype(_f32)
    s_idx = jnp.where(excl, 64.0, ns_s)
    r_idx = jnp.where(excl, 64.0, nr_s)
    ohs = (iota_v == s_idx[:, :, None]).astype(_f32)
    ohr = (iota_v == r_idx[:, :, None]).astype(_f32)
    deg2 = jnp.maximum(jnp.sum(ohr, axis=1)[:, :64], 1.0)[:, :, None]

    nodes_f = sorted_nodes
    edges_f = sorted_edges
    for we, be, wn, bn in ((fwe0, fbe0, fwn0, fbn0),
                           (fwe1, fbe1, fwn1, fbn1)):
        pad = jnp.zeros((G, VN - 64, nodes_f.shape[2]), _f32)
        nodes_pad = jnp.concatenate([nodes_f, pad], axis=1)
        gs = _bmm_hi(ohs, nodes_pad)
        gr = _bmm_hi(ohr, nodes_pad)
        edges_f = _dense(jnp.concatenate([edges_f, gs, gr], axis=2), we, be)
        agg = _bmm_t_hi(ohr, edges_f)[:, :64, :] / deg2
        nodes_f = _dense(jnp.concatenate([nodes_f, agg], axis=2), wn, bn)

    e1s = fbe0[...]
    n1s = jnp.dot(jnp.concatenate([jnp.zeros((1, NNF), _f32), e1s], axis=1),
                  fwn0[...], preferred_element_type=_f32) + fbn0[...]
    e2s = jnp.dot(jnp.concatenate([e1s, n1s, n1s], axis=1),
                  fwe1[...], preferred_element_type=_f32) + fbe1[...]
    n2s = jnp.dot(jnp.concatenate([n1s, e2s], axis=1),
                  fwn1[...], preferred_element_type=_f32) + fbn1[...]
    kf3 = keep[:, :, None]
    edges_f = edges_f * kf3 + e2s[None, :, :] * (1.0 - kf3)
    is_last = jnp.where(pl.program_id(0) == pl.num_programs(0) - 1, 1.0, 0.0)
    ig = lax.broadcasted_iota(jnp.int32, (G, 64, 1), 0)
    inn = lax.broadcasted_iota(jnp.int32, (G, 64, 1), 1)
    pmf = ((ig == G - 1) & (inn == 63)).astype(_f32) * is_last
    nodes_f = nodes_f * (1.0 - pmf) + n2s[None, :, :] * pmf

    nodes_out[...] = nodes_f.reshape(G * 64, 64)
    edges_out[...] = edges_f.reshape(G * 128, 64)
    base = (pl.program_id(0) * G
            + lax.broadcasted_iota(jnp.int32, (G, 128), 0)) * MAX_NODES
    snd_out[...] = jnp.where(excl, NUM_NODES - 1, base + ns_s.astype(jnp.int32))
    rcv_out[...] = jnp.where(excl, NUM_NODES - 1, base + nr_s.astype(jnp.int32))


def _seg_mean(edges, recv):
    agg = jax.ops.segment_sum(edges, recv, num_segments=NUM_NODES)
    deg = jax.ops.segment_sum(jnp.ones((edges.shape[0],), _f32), recv,
                              num_segments=NUM_NODES)
    return agg / jnp.maximum(deg, 1.0)[:, None]


def kernel(x, params):
    ws, gs = _flatten_params(params)
    (pw0, pb0, pn0, pnb0), (pw1, pb1, pn1, pnb1), (pw2, pb2, pn2, pnb2), \
        (fw0, fb0, fn0, fnb0), (fw1, fb1, fn1, fnb1) = gs
    f = jax.ShapeDtypeStruct

    logits, feats, init_nodes = _call(
        _mlp3_body, 1,
        [pl.BlockSpec((G, LATENT), lambda i: (i, 0))], ws,
        [pl.BlockSpec((G, 8192), lambda i: (i, 0)),
         pl.BlockSpec((G, 768), lambda i: (i, 0)),
         pl.BlockSpec((G, 64, NNF), lambda i: (i, 0, 0))],
        [f((B, 8192), _f32), f((B, 768), _f32), f((B, 64, NNF), _f32)],
    )(x, *ws)

    soft = jax.nn.softmax(logits.reshape(B, MAX_EDGES, MAX_NODES), axis=2)

    i1, i2, init_edges, edges = _call(
        _top2_body, 3,
        [pl.BlockSpec((G, 128, 64), lambda i: (i, 0, 0)),
         pl.BlockSpec((G, 768), lambda i: (i, 0)),
         pl.BlockSpec((G, 64, NNF), lambda i: (i, 0, 0))], (pw0, pb0),
        [pl.BlockSpec((G, 128), lambda i: (i, 0)),
         pl.BlockSpec((G, 128), lambda i: (i, 0)),
         pl.BlockSpec((G * 128, NEF), lambda i: (i, 0)),
         pl.BlockSpec((G * 128, 128), lambda i: (i, 0))],
        [f((B, 128), jnp.int32), f((B, 128), jnp.int32),
         f((B * 128, NEF), _f32), f((B * 128, 128), _f32)],
    )(soft, feats, init_nodes, pw0, pb0)

    recv = (i2 + jnp.arange(B, dtype=jnp.int32)[:, None] * 64).reshape(-1)
    nodes = init_nodes
    for wn, bn, we, be, eo_d in ((pn0, pnb0, pw1, pb1, 128),
                                 (pn1, pnb1, pw2, pb2, 1)):
        agg = _seg_mean(edges, recv).reshape(B, 64, -1)
        nd = nodes.shape[2]
        nodes, edges = _call(
            _round_body, 5,
            [pl.BlockSpec((G, 64, nd), lambda i: (i, 0, 0)),
             pl.BlockSpec((G, 64, agg.shape[2]), lambda i: (i, 0, 0)),
             pl.BlockSpec((G * 128, edges.shape[1]), lambda i: (i, 0)),
             pl.BlockSpec((G, 128), lambda i: (i, 0)),
             pl.BlockSpec((G, 128), lambda i: (i, 0))], (wn, bn, we, be),
            [pl.BlockSpec((G, 64, 128), lambda i: (i, 0, 0)),
             pl.BlockSpec((G * 128, eo_d), lambda i: (i, 0))],
            [f((B, 64, 128), _f32), f((B * 128, eo_d), _f32)],
        )(nodes, agg, edges, i1, i2, wn, bn, we, be)

    agg = _seg_mean(edges, recv).reshape(B, 64, 1)
    prob_nodes = _call(
        _nodelast_body, 2,
        [pl.BlockSpec((G, 64, 128), lambda i: (i, 0, 0)),
         pl.BlockSpec((G, 64, 1), lambda i: (i, 0, 0))], (pn2, pnb2),
        pl.BlockSpec((G, 64, 1), lambda i: (i, 0, 0)),
        f((B, 64, 1), _f32),
    )(nodes, agg, pn2, pnb2)

    node_probs = jax.nn.softmax(prob_nodes.reshape(B, 64), axis=1) + 1
    nn3 = jnp.rint(x[:, LATENT - 2]).reshape(B, 1, 1)
    ne3 = jnp.rint(x[:, LATENT - 1]).reshape(B, 1, 1)

    fn, fe, snd, rcv_o = _call(
        _final_body, 7,
        [pl.BlockSpec((G, 64), lambda i: (i, 0)),
         pl.BlockSpec((G, 128), lambda i: (i, 0)),
         pl.BlockSpec((G, 128), lambda i: (i, 0)),
         pl.BlockSpec((G, 64, NNF), lambda i: (i, 0, 0)),
         pl.BlockSpec((G * 128, NEF), lambda i: (i, 0)),
         pl.BlockSpec((G, 1, 1), lambda i: (i, 0, 0)),
         pl.BlockSpec((G, 1, 1), lambda i: (i, 0, 0))],
        (fw0, fb0, fn0, fnb0, fw1, fb1, fn1, fnb1),
        [pl.BlockSpec((G * 64, 64), lambda i: (i, 0)),
         pl.BlockSpec((G * 128, 64), lambda i: (i, 0)),
         pl.BlockSpec((G, 128), lambda i: (i, 0)),
         pl.BlockSpec((G, 128), lambda i: (i, 0))],
        [f((B * 64, 64), _f32), f((B * 128, 64), _f32),
         f((B, 128), jnp.int32), f((B, 128), jnp.int32)],
    )(node_probs, i1, i2, init_nodes, init_edges, nn3, ne3,
      fw0, fb0, fn0, fnb0, fw1, fb1, fn1, fnb1)

    return fn, fe, snd.reshape(-1), rcv_o.reshape(-1)


# speedup vs baseline: 3.1743x; 1.0989x over previous
import jax
import jax.numpy as jnp
from jax import lax
from jax.experimental import pallas as pl

B = 1024
LATENT = 256
MAX_NODES = 64
MAX_EDGES = 128
NEF = 8
NNF = 16
NUM_NODES = B * MAX_NODES
G = 32
VN = 72

_f32 = jnp.float32
_HI = lax.Precision.HIGHEST


def _flatten_params(params):
    ws = []
    for name in ('edge_mlp', 'feature_mlp', 'node_mlp'):
        for W, b in params[name]:
            ws.append(W)
            ws.append(b.reshape(1, -1))
    gs = []
    for name in ('prob_gnn', 'final_gnn'):
        p = params[name]
        for ep, npar in zip(p['edge'], p['node']):
            We, be = ep[0]
            Wn, bn = npar[0]
            gs.append((We, be.reshape(1, -1), Wn, bn.reshape(1, -1)))
    return ws, gs


def _bmm_hi(a, b):
    return lax.dot_general(a, b, (((2,), (1,)), ((0,), (0,))),
                           precision=_HI, preferred_element_type=_f32)


def _bmm_t_hi(a, b):
    return lax.dot_general(a, b, (((1,), (1,)), ((0,), (0,))),
                           precision=_HI, preferred_element_type=_f32)


def _dense(v, w, bias):
    g, m, k = v.shape
    r = jnp.dot(v.reshape(g * m, k), w[...],
                preferred_element_type=_f32) + bias[...]
    return r.reshape(g, m, r.shape[-1])


def _call(body, n_in_blocked, blocked_specs, weights, out_specs, out_shape):
    def wrap(*args):
        in_specs = list(blocked_specs)
        for w in args[n_in_blocked:]:
            in_specs.append(
                pl.BlockSpec(w.shape, lambda i, nd=w.ndim: (0,) * nd))
        return pl.pallas_call(
            body, grid=(B // G,), in_specs=in_specs,
            out_specs=out_specs, out_shape=out_shape)(*args)
    return wrap


def _mlp3_body(x_ref, w1e, b1e, w2e, b2e, w1f, b1f, w2f, b2f,
               w1n, b1n, w2n, b2n, lo, fo, no):
    x = x_ref[...]
    h = jnp.maximum(jnp.dot(x, w1e[...], preferred_element_type=_f32)
                    + b1e[...], 0.0)
    lo[...] = jnp.dot(h, w2e[...], preferred_element_type=_f32) + b2e[...]
    h = jnp.maximum(jnp.dot(x, w1f[...], preferred_element_type=_f32)
                    + b1f[...], 0.0)
    fo[...] = jnp.dot(h, w2f[...], preferred_element_type=_f32) + b2f[...]
    h = jnp.maximum(jnp.dot(x, w1n[...], preferred_element_type=_f32)
                    + b1n[...], 0.0)
    n = jnp.dot(h, w2n[...], preferred_element_type=_f32) + b2n[...]
    no[...] = n.reshape(G, MAX_NODES, NNF)


def _onehots(i1, i2):
    iota64 = lax.broadcasted_iota(jnp.int32, (G, MAX_EDGES, MAX_NODES), 2)
    oh1 = (iota64 == i1[:, :, None]).astype(_f32)
    oh2 = (iota64 == i2[:, :, None]).astype(_f32)
    return oh1, oh2


def _top2_body(soft_ref, feats_ref, nodes_ref, we, be,
               i1o, i2o, ieo, e1o):
    soft = soft_ref[...]
    iota64 = lax.broadcasted_iota(jnp.int32, (G, MAX_EDGES, MAX_NODES), 2)
    i1 = jnp.argmax(soft, axis=2)
    oh1 = (iota64 == i1[:, :, None]).astype(_f32)
    v13 = jnp.max(soft, axis=2, keepdims=True)
    soft2 = jnp.where(oh1 > 0, -1e30, soft)
    i2 = jnp.argmax(soft2, axis=2)
    oh2 = (iota64 == i2[:, :, None]).astype(_f32)
    v23 = jnp.max(soft2, axis=2, keepdims=True)
    i1o[...] = i1
    i2o[...] = i2
    feats = feats_ref[...].reshape(G, MAX_EDGES, NEF - 2)
    init_edges = jnp.concatenate([v13, v23, feats], axis=2)
    ieo[...] = init_edges.reshape(G * MAX_EDGES, NEF)
    nodes = nodes_ref[...]
    e_in = jnp.concatenate(
        [init_edges, _bmm_hi(oh1, nodes), _bmm_hi(oh2, nodes)], axis=2)
    e1o[...] = _dense(e_in, we, be).reshape(G * MAX_EDGES, -1)


def _round_body(nodes_ref, agg_ref, edges_ref, i1_ref, i2_ref,
                wn, bn, we, be, no, eo):
    n3 = _dense(jnp.concatenate([nodes_ref[...], agg_ref[...]], axis=2),
                wn, bn)
    no[...] = n3
    oh1, oh2 = _onehots(i1_ref[...], i2_ref[...])
    e = edges_ref[...].reshape(G, MAX_EDGES, -1)
    e_in = jnp.concatenate([e, _bmm_hi(oh1, n3), _bmm_hi(oh2, n3)], axis=2)
    eo[...] = _dense(e_in, we, be).reshape(G * MAX_EDGES, -1)


def _nodelast_body(nodes_ref, agg_ref, wn, bn, o):
    o[...] = _dense(jnp.concatenate([nodes_ref[...], agg_ref[...]], axis=2),
                    wn, bn)


def _final_body(s_ref, i1_ref, i2_ref, nodes_ref, edges_ref, nn_ref, ne_ref,
                fwe0, fbe0, fwn0, fbn0, fwe1, fbe1, fwn1, fbn1,
                nodes_out, edges_out, snd_out, rcv_out):
    s = s_ref[...]
    gt = s[:, None, :] > s[:, :, None]
    ii = lax.broadcasted_iota(jnp.int32, (G, 64, 64), 1)
    jj = lax.broadcasted_iota(jnp.int32, (G, 64, 64), 2)
    tie = (s[:, None, :] == s[:, :, None]) & (jj < ii)
    rank = jnp.sum((gt | tie).astype(_f32), axis=2)
    n_node = nn_ref[...][:, 0, 0]
    n_edge = ne_ref[...][:, 0, 0]
    init_nodes = nodes_ref[...]
    init_edges = edges_ref[...].reshape(G, MAX_EDGES, NEF)
    oh1, oh2 = _onehots(i1_ref[...], i2_ref[...])

    iota_r = lax.broadcasted_iota(jnp.int32, (G, 64, 64), 2).astype(_f32)
    perm = (iota_r == rank[:, :, None]).astype(_f32)
    sorted_nodes = _bmm_t_hi(perm, init_nodes)
    iota_n = lax.broadcasted_iota(jnp.int32, (G, 64, 1), 1).astype(_f32)
    sorted_nodes = sorted_nodes * (iota_n < n_node[:, None, None]).astype(_f32)

    ns = _bmm_hi(oh1, rank[:, :, None])[:, :, 0]
    nr = _bmm_hi(oh2, rank[:, :, None])[:, :, 0]
    logic = ((ns < n_node[:, None]) & (nr < n_node[:, None])).astype(_f32)
    n_inc = jnp.sum(logic, axis=1)

    ja = lax.broadcasted_iota(jnp.int32, (MAX_EDGES, MAX_EDGES), 0).astype(_f32)
    jb = lax.broadcasted_iota(jnp.int32, (MAX_EDGES, MAX_EDGES), 1).astype(_f32)
    c_inc_gt = jnp.dot(logic, (ja > jb).astype(_f32), precision=_HI,
                       preferred_element_type=_f32)
    c_exc_lt = jnp.dot(1.0 - logic, (ja < jb).astype(_f32), precision=_HI,
                       preferred_element_type=_f32)
    pos = jnp.where(logic > 0, c_inc_gt, n_inc[:, None] + c_exc_lt)
    iota_p = lax.broadcasted_iota(jnp.int32, (G, 128, 128), 2).astype(_f32)
    q = (iota_p == pos[:, :, None]).astype(_f32)
    sorted_edges = _bmm_t_hi(q, init_edges)
    ns_s = _bmm_t_hi(q, ns[:, :, None])[:, :, 0]
    nr_s = _bmm_t_hi(q, nr[:, :, None])[:, :, 0]

    n_edge_f = jnp.minimum(n_edge, n_inc)
    iota_e = lax.broadcasted_iota(jnp.int32, (G, 128), 1).astype(_f32)
    excl = iota_e >= n_edge_f[:, None]
    keep = (~excl).astype(_f32)
    sorted_edges = sorted_edges * keep[:, :, None]

    iota_v = lax.broadcasted_iota(jnp.int32, (G, 128, VN), 2).astype(_f32)
    s_idx = jnp.where(excl, 64.0, ns_s)
    r_idx = jnp.where(excl, 64.0, nr_s)
    ohs = (iota_v == s_idx[:, :, None]).astype(_f32)
    ohr = (iota_v == r_idx[:, :, None]).astype(_f32)
    deg2 = jnp.maximum(jnp.sum(ohr, axis=1)[:, :64], 1.0)[:, :, None]

    nodes_f = sorted_nodes
    edges_f = sorted_edges
    for we, be, wn, bn in ((fwe0, fbe0, fwn0, fbn0),
                           (fwe1, fbe1, fwn1, fbn1)):
        pad = jnp.zeros((G, VN - 64, nodes_f.shape[2]), _f32)
        nodes_pad = jnp.concatenate([nodes_f, pad], axis=1)
        gs = _bmm_hi(ohs, nodes_pad)
        gr = _bmm_hi(ohr, nodes_pad)
        edges_f = _dense(jnp.concatenate([edges_f, gs, gr], axis=2), we, be)
        agg = _bmm_t_hi(ohr, edges_f)[:, :64, :] / deg2
        nodes_f = _dense(jnp.concatenate([nodes_f, agg], axis=2), wn, bn)

    e1s = fbe0[...]
    n1s = jnp.dot(jnp.concatenate([jnp.zeros((1, NNF), _f32), e1s], axis=1),
                  fwn0[...], preferred_element_type=_f32) + fbn0[...]
    e2s = jnp.dot(jnp.concatenate([e1s, n1s, n1s], axis=1),
                  fwe1[...], preferred_element_type=_f32) + fbe1[...]
    n2s = jnp.dot(jnp.concatenate([n1s, e2s], axis=1),
                  fwn1[...], preferred_element_type=_f32) + fbn1[...]
    kf3 = keep[:, :, None]
    edges_f = edges_f * kf3 + e2s[None, :, :] * (1.0 - kf3)
    is_last = jnp.where(pl.program_id(0) == pl.num_programs(0) - 1, 1.0, 0.0)
    ig = lax.broadcasted_iota(jnp.int32, (G, 64, 1), 0)
    inn = lax.broadcasted_iota(jnp.int32, (G, 64, 1), 1)
    pmf = ((ig == G - 1) & (inn == 63)).astype(_f32) * is_last
    nodes_f = nodes_f * (1.0 - pmf) + n2s[None, :, :] * pmf

    nodes_out[...] = nodes_f.reshape(G * 64, 64)
    edges_out[...] = edges_f.reshape(G * 128, 64)
    base = (pl.program_id(0) * G
            + lax.broadcasted_iota(jnp.int32, (G, 128), 0)) * MAX_NODES
    snd_out[...] = jnp.where(excl, NUM_NODES - 1, base + ns_s.astype(jnp.int32))
    rcv_out[...] = jnp.where(excl, NUM_NODES - 1, base + nr_s.astype(jnp.int32))


def _seg_mean(edges, recv):
    agg = jax.ops.segment_sum(edges, recv, num_segments=NUM_NODES)
    deg = jax.ops.segment_sum(jnp.ones((edges.shape[0],), _f32), recv,
                              num_segments=NUM_NODES)
    return agg / jnp.maximum(deg, 1.0)[:, None]


def kernel(x, params):
    ws, gs = _flatten_params(params)
    (pw0, pb0, pn0, pnb0), (pw1, pb1, pn1, pnb1), (pw2, pb2, pn2, pnb2), \
        (fw0, fb0, fn0, fnb0), (fw1, fb1, fn1, fnb1) = gs
    f = jax.ShapeDtypeStruct

    logits, feats, init_nodes = _call(
        _mlp3_body, 1,
        [pl.BlockSpec((G, LATENT), lambda i: (i, 0))], ws,
        [pl.BlockSpec((G, 8192), lambda i: (i, 0)),
         pl.BlockSpec((G, 768), lambda i: (i, 0)),
         pl.BlockSpec((G, 64, NNF), lambda i: (i, 0, 0))],
        [f((B, 8192), _f32), f((B, 768), _f32), f((B, 64, NNF), _f32)],
    )(x, *ws)

    soft = jax.nn.softmax(logits.reshape(B, MAX_EDGES, MAX_NODES), axis=2)

    i1, i2, init_edges, edges = _call(
        _top2_body, 3,
        [pl.BlockSpec((G, 128, 64), lambda i: (i, 0, 0)),
         pl.BlockSpec((G, 768), lambda i: (i, 0)),
         pl.BlockSpec((G, 64, NNF), lambda i: (i, 0, 0))], (pw0, pb0),
        [pl.BlockSpec((G, 128), lambda i: (i, 0)),
         pl.BlockSpec((G, 128), lambda i: (i, 0)),
         pl.BlockSpec((G * 128, NEF), lambda i: (i, 0)),
         pl.BlockSpec((G * 128, 128), lambda i: (i, 0))],
        [f((B, 128), jnp.int32), f((B, 128), jnp.int32),
         f((B * 128, NEF), _f32), f((B * 128, 128), _f32)],
    )(soft, feats, init_nodes, pw0, pb0)

    recv = (i2 + jnp.arange(B, dtype=jnp.int32)[:, None] * 64).reshape(-1)
    nodes = init_nodes
    for wn, bn, we, be, eo_d in ((pn0, pnb0, pw1, pb1, 128),
                                 (pn1, pnb1, pw2, pb2, 1)):
        agg = _seg_mean(edges, recv).reshape(B, 64, -1)
        nd = nodes.shape[2]
        nodes, edges = _call(
            _round_body, 5,
            [pl.BlockSpec((G, 64, nd), lambda i: (i, 0, 0)),
             pl.BlockSpec((G, 64, agg.shape[2]), lambda i: (i, 0, 0)),
             pl.BlockSpec((G * 128, edges.shape[1]), lambda i: (i, 0)),
             pl.BlockSpec((G, 128), lambda i: (i, 0)),
             pl.BlockSpec((G, 128), lambda i: (i, 0))], (wn, bn, we, be),
            [pl.BlockSpec((G, 64, 128), lambda i: (i, 0, 0)),
             pl.BlockSpec((G * 128, eo_d), lambda i: (i, 0))],
            [f((B, 64, 128), _f32), f((B * 128, eo_d), _f32)],
        )(nodes, agg, edges, i1, i2, wn, bn, we, be)

    agg = _seg_mean(edges, recv).reshape(B, 64, 1)
    prob_nodes = _call(
        _nodelast_body, 2,
        [pl.BlockSpec((G, 64, 128), lambda i: (i, 0, 0)),
         pl.BlockSpec((G, 64, 1), lambda i: (i, 0, 0))], (pn2, pnb2),
        pl.BlockSpec((G, 64, 1), lambda i: (i, 0, 0)),
        f((B, 64, 1), _f32),
    )(nodes, agg, pn2, pnb2)

    node_probs = jax.nn.softmax(prob_nodes.reshape(B, 64), axis=1) + 1
    nn3 = jnp.rint(x[:, LATENT - 2]).reshape(B, 1, 1)
    ne3 = jnp.rint(x[:, LATENT - 1]).reshape(B, 1, 1)

    fn, fe, snd, rcv_o = _call(
        _final_body, 7,
        [pl.BlockSpec((G, 64), lambda i: (i, 0)),
         pl.BlockSpec((G, 128), lambda i: (i, 0)),
         pl.BlockSpec((G, 128), lambda i: (i, 0)),
         pl.BlockSpec((G, 64, NNF), lambda i: (i, 0, 0)),
         pl.BlockSpec((G * 128, NEF), lambda i: (i, 0)),
         pl.BlockSpec((G, 1, 1), lambda i: (i, 0, 0)),
         pl.BlockSpec((G, 1, 1), lambda i: (i, 0, 0))],
        (fw0, fb0, fn0, fnb0, fw1, fb1, fn1, fnb1),
        [pl.BlockSpec((G * 64, 64), lambda i: (i, 0)),
         pl.BlockSpec((G * 128, 64), lambda i: (i, 0)),
         pl.BlockSpec((G, 128), lambda i: (i, 0)),
         pl.BlockSpec((G, 128), lambda i: (i, 0))],
        [f((B * 64, 64), _f32), f((B * 128, 64), _f32),
         f((B, 128), jnp.int32), f((B, 128), jnp.int32)],
    )(node_probs, i1, i2, init_nodes, init_edges, nn3, ne3,
      fw0, fb0, fn0, fnb0, fw1, fb1, fn1, fnb1)

    return fn, fe, snd.reshape(-1), rcv_o.reshape(-1)
